# Initial kernel scaffold; baseline (speedup 1.0000x reference)
import jax
import jax.numpy as jnp
from jax.experimental import pallas as pl


def kernel(x, adj, W0, b0, W1, b1, W2, b2, W3, b3, g0, be0, g1, be1, g2, be2, g3, be3):
    raise NotImplementedError("write your pallas kernel here")


# speedup vs baseline: 1.1384x; 1.1384x over previous
import functools

import jax
import jax.numpy as jnp
import numpy as np
from jax.experimental import pallas as pl
from jax.experimental.pallas import tpu as pltpu

EPS = 1e-5
RB = 200


def _gcn_body(adj_ref, x_ref, W_ref, P_ref, out_ref, h_ref, s_ref):
    l = pl.program_id(0)
    r = pl.program_id(1)
    nr = pl.num_programs(1)

    @pl.when(jnp.logical_and(l == 0, r == 0))
    def _():
        h_ref[...] = x_ref[...]

    @pl.when(r == 0)
    def _():
        s_ref[...] = jnp.dot(
            h_ref[...], W_ref[0], preferred_element_type=jnp.float32
        )

    b = P_ref[0, 0, :]
    g = P_ref[0, 1, :]
    be = P_ref[0, 2, :]

    acc = jnp.dot(adj_ref[...], s_ref[...], preferred_element_type=jnp.float32)
    y = acc + b[None, :]
    y = y * (1.0 / np.sqrt(1.0 + EPS))
    mu = jnp.mean(y, axis=-1, keepdims=True)
    var = jnp.mean((y - mu) ** 2, axis=-1, keepdims=True)
    y = (y - mu) / jnp.sqrt(var + EPS) * g[None, :] + be[None, :]

    rb = adj_ref.shape[0]
    rows = pl.ds(r * rb, rb)

    @pl.when(l == 0)
    def _():
        hnew = jnp.maximum(y, 0.0)
        h_ref[rows, :] = hnew
        out_ref[...] = hnew

    @pl.when(jnp.logical_and(l >= 1, l <= 2))
    def _():
        hnew = jnp.maximum(y, 0.0) + 0.8 * h_ref[rows, :]
        h_ref[rows, :] = hnew
        out_ref[...] = hnew

    @pl.when(l == 3)
    def _():
        out_ref[...] = y + 0.2 * x_ref[rows, :]


@jax.jit
def kernel(x, adj, W0, b0, W1, b1, W2, b2, W3, b3, g0, be0, g1, be1, g2, be2,
           g3, be3):
    n, f = x.shape
    Ws = jnp.stack([W0, W1, W2, W3])
    Ps = jnp.stack(
        [jnp.stack([b0, g0, be0]), jnp.stack([b1, g1, be1]),
         jnp.stack([b2, g2, be2]), jnp.stack([b3, g3, be3])]
    )

    nr = n // RB
    grid = (4, nr)
    return pl.pallas_call(
        _gcn_body,
        grid=grid,
        in_specs=[
            pl.BlockSpec((RB, n), lambda l, r: (r, 0)),
            pl.BlockSpec((n, f), lambda l, r: (0, 0)),
            pl.BlockSpec((1, f, f), lambda l, r: (l, 0, 0)),
            pl.BlockSpec((1, 3, f), lambda l, r: (l, 0, 0)),
        ],
        out_specs=pl.BlockSpec((RB, f), lambda l, r: (r, 0)),
        out_shape=jax.ShapeDtypeStruct((n, f), jnp.float32),
        scratch_shapes=[
            pltpu.VMEM((n, f), jnp.float32),
            pltpu.VMEM((n, f), jnp.float32),
        ],
        compiler_params=pltpu.CompilerParams(
            dimension_semantics=("arbitrary", "arbitrary"),
        ),
    )(adj, x, Ws, Ps)


# speedup vs baseline: 1.3373x; 1.1747x over previous
import jax
import jax.numpy as jnp
import numpy as np
from jax.experimental import pallas as pl
from jax.experimental.pallas import tpu as pltpu

EPS = 1e-5
RB0 = 200
RB = 400


def _layer0_body(adj_ref, x_ref, W_ref, P_ref, h_ref, adjbf_ref, s_ref):
    r = pl.program_id(0)

    @pl.when(r == 0)
    def _():
        s_ref[...] = jnp.dot(
            x_ref[...], W_ref[...], preferred_element_type=jnp.float32
        ).astype(jnp.bfloat16)

    a_bf = adj_ref[...].astype(jnp.bfloat16)
    adjbf_ref[...] = a_bf
    acc = jnp.dot(a_bf, s_ref[...], preferred_element_type=jnp.float32)

    b = P_ref[0, :]
    g = P_ref[1, :]
    be = P_ref[2, :]
    y = acc + b[None, :]
    y = y * (1.0 / np.sqrt(1.0 + EPS))
    mu = jnp.mean(y, axis=-1, keepdims=True)
    var = jnp.mean((y - mu) ** 2, axis=-1, keepdims=True)
    y = (y - mu) / jnp.sqrt(var + EPS) * g[None, :] + be[None, :]
    h_ref[...] = jnp.maximum(y, 0.0)


def _layers123_body(adj_ref, x_ref, h0_ref, W_ref, P_ref, out_ref, h_ref,
                    s_ref):
    l = pl.program_id(0)
    r = pl.program_id(1)

    @pl.when(jnp.logical_and(l == 0, r == 0))
    def _():
        h_ref[...] = h0_ref[...]

    @pl.when(r == 0)
    def _():
        s_ref[...] = jnp.dot(
            h_ref[...], W_ref[0], preferred_element_type=jnp.float32
        ).astype(jnp.bfloat16)

    acc = jnp.dot(adj_ref[...], s_ref[...], preferred_element_type=jnp.float32)

    b = P_ref[0, 0, :]
    g = P_ref[0, 1, :]
    be = P_ref[0, 2, :]
    y = acc + b[None, :]
    y = y * (1.0 / np.sqrt(1.0 + EPS))
    mu = jnp.mean(y, axis=-1, keepdims=True)
    var = jnp.mean((y - mu) ** 2, axis=-1, keepdims=True)
    y = (y - mu) / jnp.sqrt(var + EPS) * g[None, :] + be[None, :]

    rb = adj_ref.shape[0]
    rows = pl.ds(r * rb, rb)

    @pl.when(l <= 1)
    def _():
        hnew = jnp.maximum(y, 0.0) + 0.8 * h_ref[rows, :]
        h_ref[rows, :] = hnew
        out_ref[...] = hnew

    @pl.when(l == 2)
    def _():
        out_ref[...] = y + 0.2 * x_ref[rows, :]


@jax.jit
def kernel(x, adj, W0, b0, W1, b1, W2, b2, W3, b3, g0, be0, g1, be1, g2, be2,
           g3, be3):
    n, f = x.shape
    P0 = jnp.stack([b0, g0, be0])
    Ws = jnp.stack([W1, W2, W3])
    Ps = jnp.stack(
        [jnp.stack([b1, g1, be1]), jnp.stack([b2, g2, be2]),
         jnp.stack([b3, g3, be3])]
    )

    h1, adj_bf = pl.pallas_call(
        _layer0_body,
        grid=(n // RB0,),
        in_specs=[
            pl.BlockSpec((RB0, n), lambda r: (r, 0)),
            pl.BlockSpec((n, f), lambda r: (0, 0)),
            pl.BlockSpec((f, f), lambda r: (0, 0)),
            pl.BlockSpec((3, f), lambda r: (0, 0)),
        ],
        out_specs=[
            pl.BlockSpec((RB0, f), lambda r: (r, 0)),
            pl.BlockSpec((RB0, n), lambda r: (r, 0)),
        ],
        out_shape=[
            jax.ShapeDtypeStruct((n, f), jnp.float32),
            jax.ShapeDtypeStruct((n, n), jnp.bfloat16),
        ],
        scratch_shapes=[pltpu.VMEM((n, f), jnp.bfloat16)],
        compiler_params=pltpu.CompilerParams(
            dimension_semantics=("arbitrary",),
        ),
    )(adj, x, W0, P0)

    return pl.pallas_call(
        _layers123_body,
        grid=(3, n // RB),
        in_specs=[
            pl.BlockSpec((RB, n), lambda l, r: (r, 0)),
            pl.BlockSpec((n, f), lambda l, r: (0, 0)),
            pl.BlockSpec((n, f), lambda l, r: (0, 0)),
            pl.BlockSpec((1, f, f), lambda l, r: (l, 0, 0)),
            pl.BlockSpec((1, 3, f), lambda l, r: (l, 0, 0)),
        ],
        out_specs=pl.BlockSpec((RB, f), lambda l, r: (r, 0)),
        out_shape=jax.ShapeDtypeStruct((n, f), jnp.float32),
        scratch_shapes=[
            pltpu.VMEM((n, f), jnp.float32),
            pltpu.VMEM((n, f), jnp.bfloat16),
        ],
        compiler_params=pltpu.CompilerParams(
            dimension_semantics=("arbitrary", "arbitrary"),
        ),
    )(adj_bf, x, h1, Ws, Ps)


# speedup vs baseline: 1.4376x; 1.0750x over previous
import jax
import jax.numpy as jnp
import numpy as np
from jax.experimental import pallas as pl
from jax.experimental.pallas import tpu as pltpu

EPS = 1e-5
RB0 = 200
RB = 1000


def _layer0_body(adj_ref, x_ref, W_ref, P_ref, h_ref, adjbf_ref, s_ref):
    r = pl.program_id(0)

    @pl.when(r == 0)
    def _():
        s_ref[...] = jnp.dot(
            x_ref[...], W_ref[...], preferred_element_type=jnp.float32
        ).astype(jnp.bfloat16)

    a_bf = adj_ref[...].astype(jnp.bfloat16)
    adjbf_ref[...] = a_bf
    acc = jnp.dot(a_bf, s_ref[...], preferred_element_type=jnp.float32)

    b = P_ref[0, :]
    g = P_ref[1, :]
    be = P_ref[2, :]
    y = acc + b[None, :]
    y = y * (1.0 / np.sqrt(1.0 + EPS))
    mu = jnp.mean(y, axis=-1, keepdims=True)
    var = jnp.mean((y - mu) ** 2, axis=-1, keepdims=True)
    y = (y - mu) / jnp.sqrt(var + EPS) * g[None, :] + be[None, :]
    h_ref[...] = jnp.maximum(y, 0.0)


def _layers123_body(adj_ref, x_ref, h0_ref, W_ref, P_ref, out_ref, h_ref,
                    s_ref):
    l = pl.program_id(0)
    r = pl.program_id(1)

    @pl.when(jnp.logical_and(l == 0, r == 0))
    def _():
        h_ref[...] = h0_ref[...]

    @pl.when(r == 0)
    def _():
        s_ref[...] = jnp.dot(
            h_ref[...], W_ref[0], preferred_element_type=jnp.float32
        ).astype(jnp.bfloat16)

    acc = jnp.dot(adj_ref[...], s_ref[...], preferred_element_type=jnp.float32)

    b = P_ref[0, 0, :]
    g = P_ref[0, 1, :]
    be = P_ref[0, 2, :]
    y = acc + b[None, :]
    y = y * (1.0 / np.sqrt(1.0 + EPS))
    mu = jnp.mean(y, axis=-1, keepdims=True)
    var = jnp.mean((y - mu) ** 2, axis=-1, keepdims=True)
    y = (y - mu) / jnp.sqrt(var + EPS) * g[None, :] + be[None, :]

    rb = adj_ref.shape[0]
    rows = pl.ds(r * rb, rb)

    @pl.when(l <= 1)
    def _():
        hnew = jnp.maximum(y, 0.0) + 0.8 * h_ref[rows, :]
        h_ref[rows, :] = hnew
        out_ref[...] = hnew

    @pl.when(l == 2)
    def _():
        out_ref[...] = y + 0.2 * x_ref[rows, :]


@jax.jit
def kernel(x, adj, W0, b0, W1, b1, W2, b2, W3, b3, g0, be0, g1, be1, g2, be2,
           g3, be3):
    n, f = x.shape
    P0 = jnp.stack([b0, g0, be0])
    Ws = jnp.stack([W1, W2, W3])
    Ps = jnp.stack(
        [jnp.stack([b1, g1, be1]), jnp.stack([b2, g2, be2]),
         jnp.stack([b3, g3, be3])]
    )

    h1, adj_bf = pl.pallas_call(
        _layer0_body,
        grid=(n // RB0,),
        in_specs=[
            pl.BlockSpec((RB0, n), lambda r: (r, 0)),
            pl.BlockSpec((n, f), lambda r: (0, 0)),
            pl.BlockSpec((f, f), lambda r: (0, 0)),
            pl.BlockSpec((3, f), lambda r: (0, 0)),
        ],
        out_specs=[
            pl.BlockSpec((RB0, f), lambda r: (r, 0)),
            pl.BlockSpec((RB0, n), lambda r: (r, 0)),
        ],
        out_shape=[
            jax.ShapeDtypeStruct((n, f), jnp.float32),
            jax.ShapeDtypeStruct((n, n), jnp.bfloat16),
        ],
        scratch_shapes=[pltpu.VMEM((n, f), jnp.bfloat16)],
        compiler_params=pltpu.CompilerParams(
            dimension_semantics=("arbitrary",),
        ),
    )(adj, x, W0, P0)

    return pl.pallas_call(
        _layers123_body,
        grid=(3, n // RB),
        in_specs=[
            pl.BlockSpec((RB, n), lambda l, r: (r, 0)),
            pl.BlockSpec((n, f), lambda l, r: (0, 0)),
            pl.BlockSpec((n, f), lambda l, r: (0, 0)),
            pl.BlockSpec((1, f, f), lambda l, r: (l, 0, 0)),
            pl.BlockSpec((1, 3, f), lambda l, r: (l, 0, 0)),
        ],
        out_specs=pl.BlockSpec((RB, f), lambda l, r: (r, 0)),
        out_shape=jax.ShapeDtypeStruct((n, f), jnp.float32),
        scratch_shapes=[
            pltpu.VMEM((n, f), jnp.float32),
            pltpu.VMEM((n, f), jnp.bfloat16),
        ],
        compiler_params=pltpu.CompilerParams(
            dimension_semantics=("arbitrary", "arbitrary"),
        ),
    )(adj_bf, x, h1, Ws, Ps)


# speedup vs baseline: 1.4478x; 1.0071x over previous
import jax
import jax.numpy as jnp
import numpy as np
from jax.experimental import pallas as pl
from jax.experimental.pallas import tpu as pltpu

EPS = 1e-5
RB0 = 400
RB = 1000


def _layer0_body(adj_ref, x_ref, W_ref, P_ref, h_ref, adjbf_ref, s_ref):
    r = pl.program_id(0)

    @pl.when(r == 0)
    def _():
        s_ref[...] = jnp.dot(
            x_ref[...], W_ref[...], preferred_element_type=jnp.float32
        ).astype(jnp.bfloat16)

    a_bf = adj_ref[...].astype(jnp.bfloat16)
    adjbf_ref[...] = a_bf
    acc = jnp.dot(a_bf, s_ref[...], preferred_element_type=jnp.float32)

    b = P_ref[0, :]
    g = P_ref[1, :]
    be = P_ref[2, :]
    y = acc + b[None, :]
    y = y * (1.0 / np.sqrt(1.0 + EPS))
    mu = jnp.mean(y, axis=-1, keepdims=True)
    var = jnp.mean((y - mu) ** 2, axis=-1, keepdims=True)
    y = (y - mu) / jnp.sqrt(var + EPS) * g[None, :] + be[None, :]
    h_ref[...] = jnp.maximum(y, 0.0)


def _layers123_body(adj_ref, x_ref, h0_ref, W_ref, P_ref, out_ref, h_ref,
                    s_ref):
    l = pl.program_id(0)
    r = pl.program_id(1)

    @pl.when(jnp.logical_and(l == 0, r == 0))
    def _():
        h_ref[...] = h0_ref[...]

    @pl.when(r == 0)
    def _():
        s_ref[...] = jnp.dot(
            h_ref[...], W_ref[0], preferred_element_type=jnp.float32
        ).astype(jnp.bfloat16)

    acc = jnp.dot(adj_ref[...], s_ref[...], preferred_element_type=jnp.float32)

    b = P_ref[0, 0, :]
    g = P_ref[0, 1, :]
    be = P_ref[0, 2, :]
    y = acc + b[None, :]
    y = y * (1.0 / np.sqrt(1.0 + EPS))
    mu = jnp.mean(y, axis=-1, keepdims=True)
    var = jnp.mean((y - mu) ** 2, axis=-1, keepdims=True)
    y = (y - mu) / jnp.sqrt(var + EPS) * g[None, :] + be[None, :]

    rb = adj_ref.shape[0]
    rows = pl.ds(r * rb, rb)

    @pl.when(l <= 1)
    def _():
        hnew = jnp.maximum(y, 0.0) + 0.8 * h_ref[rows, :]
        h_ref[rows, :] = hnew
        out_ref[...] = hnew

    @pl.when(l == 2)
    def _():
        out_ref[...] = y + 0.2 * x_ref[rows, :]


@jax.jit
def kernel(x, adj, W0, b0, W1, b1, W2, b2, W3, b3, g0, be0, g1, be1, g2, be2,
           g3, be3):
    n, f = x.shape
    P0 = jnp.stack([b0, g0, be0])
    Ws = jnp.stack([W1, W2, W3])
    Ps = jnp.stack(
        [jnp.stack([b1, g1, be1]), jnp.stack([b2, g2, be2]),
         jnp.stack([b3, g3, be3])]
    )

    h1, adj_bf = pl.pallas_call(
        _layer0_body,
        grid=(n // RB0,),
        in_specs=[
            pl.BlockSpec((RB0, n), lambda r: (r, 0)),
            pl.BlockSpec((n, f), lambda r: (0, 0)),
            pl.BlockSpec((f, f), lambda r: (0, 0)),
            pl.BlockSpec((3, f), lambda r: (0, 0)),
        ],
        out_specs=[
            pl.BlockSpec((RB0, f), lambda r: (r, 0)),
            pl.BlockSpec((RB0, n), lambda r: (r, 0)),
        ],
        out_shape=[
            jax.ShapeDtypeStruct((n, f), jnp.float32),
            jax.ShapeDtypeStruct((n, n), jnp.bfloat16),
        ],
        scratch_shapes=[pltpu.VMEM((n, f), jnp.bfloat16)],
        compiler_params=pltpu.CompilerParams(
            dimension_semantics=("arbitrary",),
        ),
    )(adj, x, W0, P0)

    return pl.pallas_call(
        _layers123_body,
        grid=(3, n // RB),
        in_specs=[
            pl.BlockSpec((RB, n), lambda l, r: (r, 0)),
            pl.BlockSpec((n, f), lambda l, r: (0, 0)),
            pl.BlockSpec((n, f), lambda l, r: (0, 0)),
            pl.BlockSpec((1, f, f), lambda l, r: (l, 0, 0)),
            pl.BlockSpec((1, 3, f), lambda l, r: (l, 0, 0)),
        ],
        out_specs=pl.BlockSpec((RB, f), lambda l, r: (r, 0)),
        out_shape=jax.ShapeDtypeStruct((n, f), jnp.float32),
        scratch_shapes=[
            pltpu.VMEM((n, f), jnp.float32),
            pltpu.VMEM((n, f), jnp.bfloat16),
        ],
        compiler_params=pltpu.CompilerParams(
            dimension_semantics=("arbitrary", "arbitrary"),
        ),
    )(adj_bf, x, h1, Ws, Ps)


# speedup vs baseline: 1.4484x; 1.0004x over previous
import jax
import jax.numpy as jnp
import numpy as np
from jax.experimental import pallas as pl
from jax.experimental.pallas import tpu as pltpu

EPS = 1e-5
RB0 = 400
RB = 1000


def _layer0_body(adj_ref, x_ref, W_ref, P_ref, h_ref, adjbf_ref, s_ref):
    r = pl.program_id(0)

    @pl.when(r == 0)
    def _():
        s_ref[...] = jnp.dot(
            x_ref[...], W_ref[...], preferred_element_type=jnp.float32
        ).astype(jnp.bfloat16)

    a_bf = adj_ref[...].astype(jnp.bfloat16)
    adjbf_ref[...] = a_bf
    acc = jnp.dot(a_bf, s_ref[...], preferred_element_type=jnp.float32)

    b = P_ref[0, :]
    g = P_ref[1, :]
    be = P_ref[2, :]
    y = acc + b[None, :]
    y = y * (1.0 / np.sqrt(1.0 + EPS))
    mu = jnp.mean(y, axis=-1, keepdims=True)
    var = jnp.mean((y - mu) ** 2, axis=-1, keepdims=True)
    y = (y - mu) / jnp.sqrt(var + EPS) * g[None, :] + be[None, :]
    h_ref[...] = jnp.maximum(y, 0.0)


def _layers123_body(adj_ref, x_ref, h0_ref, W_ref, P_ref, out_ref, h_ref,
                    s_ref):
    l = pl.program_id(0)
    r = pl.program_id(1)
    nr = pl.num_programs(1)

    @pl.when(jnp.logical_and(l == 0, r == 0))
    def _():
        s_ref[...] = jnp.dot(
            h0_ref[...], W_ref[0], preferred_element_type=jnp.float32
        ).astype(jnp.bfloat16)

    acc = jnp.dot(adj_ref[...], s_ref[...], preferred_element_type=jnp.float32)

    b = P_ref[l, 0, :]
    g = P_ref[l, 1, :]
    be = P_ref[l, 2, :]
    y = acc + b[None, :]
    y = y * (1.0 / np.sqrt(1.0 + EPS))
    mu = jnp.mean(y, axis=-1, keepdims=True)
    var = jnp.mean((y - mu) ** 2, axis=-1, keepdims=True)
    y = (y - mu) / jnp.sqrt(var + EPS) * g[None, :] + be[None, :]

    rb = adj_ref.shape[0]
    rows = pl.ds(r * rb, rb)

    @pl.when(l == 0)
    def _():
        hnew = jnp.maximum(y, 0.0) + 0.8 * h0_ref[rows, :]
        h_ref[rows, :] = hnew
        out_ref[...] = hnew

    @pl.when(l == 1)
    def _():
        hnew = jnp.maximum(y, 0.0) + 0.8 * h_ref[rows, :]
        h_ref[rows, :] = hnew
        out_ref[...] = hnew

    @pl.when(l == 2)
    def _():
        out_ref[...] = y + 0.2 * x_ref[rows, :]

    @pl.when(jnp.logical_and(r == nr - 1, l < 2))
    def _():
        s_ref[...] = jnp.dot(
            h_ref[...], W_ref[l + 1], preferred_element_type=jnp.float32
        ).astype(jnp.bfloat16)


@jax.jit
def kernel(x, adj, W0, b0, W1, b1, W2, b2, W3, b3, g0, be0, g1, be1, g2, be2,
           g3, be3):
    n, f = x.shape
    P0 = jnp.stack([b0, g0, be0])
    Ws = jnp.stack([W1, W2, W3])
    Ps = jnp.stack(
        [jnp.stack([b1, g1, be1]), jnp.stack([b2, g2, be2]),
         jnp.stack([b3, g3, be3])]
    )

    h1, adj_bf = pl.pallas_call(
        _layer0_body,
        grid=(n // RB0,),
        in_specs=[
            pl.BlockSpec((RB0, n), lambda r: (r, 0)),
            pl.BlockSpec((n, f), lambda r: (0, 0)),
            pl.BlockSpec((f, f), lambda r: (0, 0)),
            pl.BlockSpec((3, f), lambda r: (0, 0)),
        ],
        out_specs=[
            pl.BlockSpec((RB0, f), lambda r: (r, 0)),
            pl.BlockSpec((RB0, n), lambda r: (r, 0)),
        ],
        out_shape=[
            jax.ShapeDtypeStruct((n, f), jnp.float32),
            jax.ShapeDtypeStruct((n, n), jnp.bfloat16),
        ],
        scratch_shapes=[pltpu.VMEM((n, f), jnp.bfloat16)],
        compiler_params=pltpu.CompilerParams(
            dimension_semantics=("arbitrary",),
        ),
    )(adj, x, W0, P0)

    return pl.pallas_call(
        _layers123_body,
        grid=(3, n // RB),
        in_specs=[
            pl.BlockSpec((RB, n), lambda l, r: (r, 0)),
            pl.BlockSpec((n, f), lambda l, r: (0, 0)),
            pl.BlockSpec((n, f), lambda l, r: (0, 0)),
            pl.BlockSpec((3, f, f), lambda l, r: (0, 0, 0)),
            pl.BlockSpec((3, 3, f), lambda l, r: (0, 0, 0)),
        ],
        out_specs=pl.BlockSpec((RB, f), lambda l, r: (r, 0)),
        out_shape=jax.ShapeDtypeStruct((n, f), jnp.float32),
        scratch_shapes=[
            pltpu.VMEM((n, f), jnp.float32),
            pltpu.VMEM((n, f), jnp.bfloat16),
        ],
        compiler_params=pltpu.CompilerParams(
            dimension_semantics=("arbitrary", "arbitrary"),
        ),
    )(adj_bf, x, h1, Ws, Ps)


# speedup vs baseline: 1.4575x; 1.0063x over previous
import jax
import jax.numpy as jnp
import numpy as np
from jax.experimental import pallas as pl
from jax.experimental.pallas import tpu as pltpu

EPS = 1e-5
RB0 = 400
RB = 1000


def _layer0_body(adj_ref, x_ref, W_ref, P_ref, h_ref, adjbf_ref, s_ref):
    r = pl.program_id(0)

    @pl.when(r == 0)
    def _():
        s_ref[...] = jnp.dot(
            x_ref[...], W_ref[...], preferred_element_type=jnp.float32
        ).astype(jnp.bfloat16)

    a_bf = adj_ref[...].astype(jnp.bfloat16)
    adjbf_ref[...] = a_bf
    acc = jnp.dot(a_bf, s_ref[...], preferred_element_type=jnp.float32)

    b = P_ref[0, :]
    g = P_ref[1, :]
    be = P_ref[2, :]
    y = acc + b[None, :]
    y = y * (1.0 / np.sqrt(1.0 + EPS))
    mu = jnp.mean(y, axis=-1, keepdims=True)
    var = jnp.mean((y - mu) ** 2, axis=-1, keepdims=True)
    y = (y - mu) / jnp.sqrt(var + EPS) * g[None, :] + be[None, :]
    h_ref[...] = jnp.maximum(y, 0.0)


def _layers123_body(adj_ref, x_ref, h0_ref, W_ref, P_ref, out_ref, h_ref,
                    s_ref):
    l = pl.program_id(0)
    r = pl.program_id(1)
    nr = pl.num_programs(1)

    @pl.when(jnp.logical_and(l == 0, r == 0))
    def _():
        s_ref[...] = jnp.dot(
            h0_ref[...], W_ref[0], preferred_element_type=jnp.float32
        ).astype(jnp.bfloat16)

    acc = jnp.dot(adj_ref[...], s_ref[...], preferred_element_type=jnp.float32)

    b = P_ref[l, 0, :]
    g = P_ref[l, 1, :]
    be = P_ref[l, 2, :]
    y = acc + b[None, :]
    y = y * (1.0 / np.sqrt(1.0 + EPS))
    mu = jnp.mean(y, axis=-1, keepdims=True)
    var = jnp.mean((y - mu) ** 2, axis=-1, keepdims=True)
    y = (y - mu) / jnp.sqrt(var + EPS) * g[None, :] + be[None, :]

    rb = adj_ref.shape[0]
    rows = pl.ds(r * rb, rb)

    @pl.when(l == 0)
    def _():
        h_ref[rows, :] = jnp.maximum(y, 0.0) + 0.8 * h0_ref[rows, :]

    @pl.when(l == 1)
    def _():
        h_ref[rows, :] = jnp.maximum(y, 0.0) + 0.8 * h_ref[rows, :]

    @pl.when(l == 2)
    def _():
        out_ref[...] = y + 0.2 * x_ref[rows, :]

    @pl.when(jnp.logical_and(r == nr - 1, l < 2))
    def _():
        s_ref[...] = jnp.dot(
            h_ref[...], W_ref[l + 1], preferred_element_type=jnp.float32
        ).astype(jnp.bfloat16)


@jax.jit
def kernel(x, adj, W0, b0, W1, b1, W2, b2, W3, b3, g0, be0, g1, be1, g2, be2,
           g3, be3):
    n, f = x.shape
    P0 = jnp.stack([b0, g0, be0])
    Ws = jnp.stack([W1, W2, W3])
    Ps = jnp.stack(
        [jnp.stack([b1, g1, be1]), jnp.stack([b2, g2, be2]),
         jnp.stack([b3, g3, be3])]
    )

    h1, adj_bf = pl.pallas_call(
        _layer0_body,
        grid=(n // RB0,),
        in_specs=[
            pl.BlockSpec((RB0, n), lambda r: (r, 0)),
            pl.BlockSpec((n, f), lambda r: (0, 0)),
            pl.BlockSpec((f, f), lambda r: (0, 0)),
            pl.BlockSpec((3, f), lambda r: (0, 0)),
        ],
        out_specs=[
            pl.BlockSpec((RB0, f), lambda r: (r, 0)),
            pl.BlockSpec((RB0, n), lambda r: (r, 0)),
        ],
        out_shape=[
            jax.ShapeDtypeStruct((n, f), jnp.float32),
            jax.ShapeDtypeStruct((n, n), jnp.bfloat16),
        ],
        scratch_shapes=[pltpu.VMEM((n, f), jnp.bfloat16)],
        compiler_params=pltpu.CompilerParams(
            dimension_semantics=("arbitrary",),
        ),
    )(adj, x, W0, P0)

    return pl.pallas_call(
        _layers123_body,
        grid=(3, n // RB),
        in_specs=[
            pl.BlockSpec((RB, n), lambda l, r: (r, 0)),
            pl.BlockSpec((n, f), lambda l, r: (0, 0)),
            pl.BlockSpec((n, f), lambda l, r: (0, 0)),
            pl.BlockSpec((3, f, f), lambda l, r: (0, 0, 0)),
            pl.BlockSpec((3, 3, f), lambda l, r: (0, 0, 0)),
        ],
        out_specs=pl.BlockSpec((RB, f), lambda l, r: ((l // 2) * r, 0)),
        out_shape=jax.ShapeDtypeStruct((n, f), jnp.float32),
        scratch_shapes=[
            pltpu.VMEM((n, f), jnp.float32),
            pltpu.VMEM((n, f), jnp.bfloat16),
        ],
        compiler_params=pltpu.CompilerParams(
            dimension_semantics=("arbitrary", "arbitrary"),
        ),
    )(adj_bf, x, h1, Ws, Ps)
